# baseline (speedup 1.0000x reference)
import jax
import jax.numpy as jnp
from jax.experimental import pallas as pl
from jax.experimental.pallas import tpu as pltpu


def _make_fused_kernel(S, H, KB, NSTEPS, DEPTH, PREFETCH):
    dn_t = (((1,), (1,)), ((), ()))

    def _fused(x_hbm, pe_ref, w_ref, b_ref, o_hbm,
               wxb_ref, c_ref, x_buf, o_buf, in_sems, out_sems):
        def dma_in(slot, step):
            pltpu.make_async_copy(
                x_hbm.at[pl.ds(step * KB, KB)], x_buf.at[slot],
                in_sems.at[slot]).start()

        def wait_in(slot):
            pltpu.make_async_copy(
                x_hbm.at[pl.ds(0, KB)], x_buf.at[slot],
                in_sems.at[slot]).wait()

        def dma_out(slot, step):
            pltpu.make_async_copy(
                o_buf.at[slot], o_hbm.at[pl.ds(step * KB, KB)],
                out_sems.at[slot]).start()

        def wait_out(slot):
            pltpu.make_async_copy(
                o_buf.at[slot], o_hbm.at[pl.ds(0, KB)],
                out_sems.at[slot]).wait()

        for s in range(PREFETCH):
            dma_in(s, s)

        wxb_ref[...] = w_ref[:, :H].astype(jnp.bfloat16)
        c_ref[...] = (
            jax.lax.dot_general(
                pe_ref[:S, :], w_ref[:, H:], dimension_numbers=dn_t,
                preferred_element_type=jnp.float32)
            + b_ref[...]
        )

        def body(step, _):
            slot = jax.lax.rem(step, DEPTH)
            wait_in(slot)

            @pl.when(step >= DEPTH)
            def _():
                wait_out(slot)

            x = x_buf[slot].reshape(KB * S, H).astype(jnp.bfloat16)
            acc = jax.lax.dot_general(
                x, wxb_ref[...], dimension_numbers=dn_t,
                preferred_element_type=jnp.float32)
            o_buf[slot] = acc.reshape(KB, S, H) + c_ref[...][None]

            dma_out(slot, step)

            @pl.when(step + PREFETCH < NSTEPS)
            def _():
                dma_in(jax.lax.rem(step + PREFETCH, DEPTH), step + PREFETCH)

            return ()

        jax.lax.fori_loop(0, NSTEPS, body, ())

        for k in range(min(DEPTH, NSTEPS)):
            wait_out(jax.lax.rem(jnp.int32(NSTEPS - 1 - k), DEPTH))

    return _fused


def kernel(inp, pe, proj_W, proj_b):
    B, S, H = inp.shape
    bias = proj_b.reshape(1, H)

    KB = 2 if B % 2 == 0 else 1
    NSTEPS = B // KB
    DEPTH = min(8, NSTEPS)
    PREFETCH = min(6, NSTEPS)

    return pl.pallas_call(
        _make_fused_kernel(S, H, KB, NSTEPS, DEPTH, PREFETCH),
        out_shape=jax.ShapeDtypeStruct((B, S, H), inp.dtype),
        in_specs=[
            pl.BlockSpec(memory_space=pl.ANY),
            pl.BlockSpec(memory_space=pltpu.VMEM),
            pl.BlockSpec(memory_space=pltpu.VMEM),
            pl.BlockSpec(memory_space=pltpu.VMEM),
        ],
        out_specs=pl.BlockSpec(memory_space=pl.ANY),
        scratch_shapes=[
            pltpu.VMEM((H, H), jnp.bfloat16),
            pltpu.VMEM((S, H), jnp.float32),
            pltpu.VMEM((DEPTH, KB, S, H), jnp.float32),
            pltpu.VMEM((DEPTH, KB, S, H), jnp.float32),
            pltpu.SemaphoreType.DMA((DEPTH,)),
            pltpu.SemaphoreType.DMA((DEPTH,)),
        ],
    )(inp, pe, proj_W, bias)


# speedup vs baseline: 1.0104x; 1.0104x over previous
import jax
import jax.numpy as jnp
from jax.experimental import pallas as pl
from jax.experimental.pallas import tpu as pltpu


def _make_fused_kernel(S, H, KB, NSTEPS, DEPTH, PREFETCH):
    dn_t = (((1,), (1,)), ((), ()))

    def _fused(x_hbm, pe_ref, w_ref, b_ref, o_hbm,
               wxb_ref, c_ref, x_buf, o_buf, in_sems, out_sems):
        def dma_in(slot, step):
            pltpu.make_async_copy(
                x_hbm.at[pl.ds(step * KB, KB)], x_buf.at[slot],
                in_sems.at[slot]).start()

        def wait_in(slot):
            pltpu.make_async_copy(
                x_hbm.at[pl.ds(0, KB)], x_buf.at[slot],
                in_sems.at[slot]).wait()

        def dma_out(slot, step):
            pltpu.make_async_copy(
                o_buf.at[slot], o_hbm.at[pl.ds(step * KB, KB)],
                out_sems.at[slot]).start()

        def wait_out(slot):
            pltpu.make_async_copy(
                o_buf.at[slot], o_hbm.at[pl.ds(0, KB)],
                out_sems.at[slot]).wait()

        for s in range(PREFETCH):
            dma_in(s, s)

        wxb_ref[...] = w_ref[:, :H].astype(jnp.bfloat16)
        c_ref[...] = (
            jax.lax.dot_general(
                pe_ref[:S, :], w_ref[:, H:], dimension_numbers=dn_t,
                preferred_element_type=jnp.float32)
            + b_ref[...]
        )

        def body(step, _):
            slot = jax.lax.rem(step, DEPTH)
            wait_in(slot)

            @pl.when(step >= DEPTH)
            def _():
                wait_out(slot)

            x = x_buf[slot].reshape(KB * S, H).astype(jnp.bfloat16)
            acc = jax.lax.dot_general(
                x, wxb_ref[...], dimension_numbers=dn_t,
                preferred_element_type=jnp.float32)
            o_buf[slot] = acc.reshape(KB, S, H) + c_ref[...][None]

            dma_out(slot, step)

            @pl.when(step + PREFETCH < NSTEPS)
            def _():
                dma_in(jax.lax.rem(step + PREFETCH, DEPTH), step + PREFETCH)

            return ()

        jax.lax.fori_loop(0, NSTEPS, body, ())

        for k in range(min(DEPTH, NSTEPS)):
            wait_out(jax.lax.rem(jnp.int32(NSTEPS - 1 - k), DEPTH))

    return _fused


def kernel(inp, pe, proj_W, proj_b):
    B, S, H = inp.shape
    bias = proj_b.reshape(1, H)

    KB = 4 if B % 4 == 0 else 1
    NSTEPS = B // KB
    DEPTH = min(5, NSTEPS)
    PREFETCH = min(4, NSTEPS)

    return pl.pallas_call(
        _make_fused_kernel(S, H, KB, NSTEPS, DEPTH, PREFETCH),
        out_shape=jax.ShapeDtypeStruct((B, S, H), inp.dtype),
        in_specs=[
            pl.BlockSpec(memory_space=pl.ANY),
            pl.BlockSpec(memory_space=pltpu.VMEM),
            pl.BlockSpec(memory_space=pltpu.VMEM),
            pl.BlockSpec(memory_space=pltpu.VMEM),
        ],
        out_specs=pl.BlockSpec(memory_space=pl.ANY),
        scratch_shapes=[
            pltpu.VMEM((H, H), jnp.bfloat16),
            pltpu.VMEM((S, H), jnp.float32),
            pltpu.VMEM((DEPTH, KB, S, H), jnp.float32),
            pltpu.VMEM((DEPTH, KB, S, H), jnp.float32),
            pltpu.SemaphoreType.DMA((DEPTH,)),
            pltpu.SemaphoreType.DMA((DEPTH,)),
        ],
    )(inp, pe, proj_W, bias)
